# baseline (speedup 1.0000x reference)
import functools

import jax
import jax.numpy as jnp
from jax import lax
from jax.experimental import pallas as pl
from jax.experimental.pallas import tpu as pltpu
from jax.experimental.pallas import tpu_sc as plsc

EPS = 1e-12


def _rowsum(vs):
    while len(vs) > 1:
        vs = [a + b for a, b in zip(vs[::2], vs[1::2])]
    return vs[0]


_TAKE_DNUMS = lax.GatherDimensionNumbers(
    offset_dims=(), collapsed_slice_dims=(0,), start_index_map=(0,))


def _lane_shuffle(x, idx):
    return lax.gather(x, idx[:, None], dimension_numbers=_TAKE_DNUMS,
                      slice_sizes=(1,),
                      mode=lax.GatherScatterMode.PROMISE_IN_BOUNDS)


def _lane_allsum(x):
    lanes = lax.iota(jnp.int32, 16)
    for d in (8, 4, 2, 1):
        x = x + _lane_shuffle(x, lanes ^ d)
    return x


def _make_fused(V, D, N, L):
    info = plsc.get_sparse_core_info()
    NC, NS = info.num_cores, info.num_subcores
    NW = NC * NS
    CH = 128
    assert N % (NW * CH) == 0
    n_chunks = N // (NW * CH)
    assert n_chunks % 2 == 0
    n_pairs = n_chunks // 2
    NK = D // 16

    mesh = plsc.VectorSubcoreMesh(core_axis_name="c", subcore_axis_name="s")

    @functools.partial(
        pl.kernel,
        mesh=mesh,
        out_type=jax.ShapeDtypeStruct((N, D), jnp.float32),
        compiler_params=pltpu.CompilerParams(needs_layout_passes=False),
        scratch_types=[
            pltpu.VMEM((n_chunks, CH), jnp.int32),
            pltpu.VMEM((L, D), jnp.float32),
            pltpu.VMEM((D,), jnp.float32),
            pltpu.VMEM((D,), jnp.float32),
            pltpu.VMEM((CH, D), jnp.float32),
            pltpu.VMEM((CH, D), jnp.float32),
            pltpu.VMEM((CH, D), jnp.float32),
            pltpu.VMEM((CH, D), jnp.float32),
            pltpu.SemaphoreType.DMA,
            pltpu.SemaphoreType.DMA,
            pltpu.SemaphoreType.DMA,
            pltpu.SemaphoreType.DMA,
        ],
    )
    def fused(W_hbm, ids_hbm, pos_hbm, g_hbm, b_hbm, out_hbm,
              idx_v, pos_v, g_v, b_v, bufA, bufB, obufA, obufB,
              gsemA, gsemB, ssemA, ssemB):
        wid = lax.axis_index("s") * NC + lax.axis_index("c")
        pltpu.sync_copy(ids_hbm.at[wid], idx_v)
        pltpu.sync_copy(pos_hbm, pos_v)
        pltpu.sync_copy(g_hbm, g_v)
        pltpu.sync_copy(b_hbm, b_v)

        def gather(c, buf, sem):
            pltpu.async_copy(W_hbm.at[idx_v.at[c]], buf, sem)

        def gather_wait(c, buf, sem):
            pltpu.make_async_copy(W_hbm.at[idx_v.at[c]], buf, sem).wait()

        def store(c, obuf, sem):
            pltpu.async_copy(
                obuf, out_hbm.at[pl.ds((wid * n_chunks + c) * CH, CH)], sem)

        def store_wait(c, obuf, sem):
            pltpu.make_async_copy(
                obuf, out_hbm.at[pl.ds((wid * n_chunks + c) * CH, CH)],
                sem).wait()

        def compute(c, buf, obuf):
            row0 = (wid * n_chunks + c) * CH

            def row_body(r, _):
                pr = lax.rem(row0 + r, L)
                xs = [buf[r, pl.ds(16 * k, 16)] + pos_v[pr, pl.ds(16 * k, 16)]
                      for k in range(NK)]
                mean = _lane_allsum(_rowsum(xs)) * (1.0 / D)
                sq = _lane_allsum(_rowsum([x * x for x in xs])) * (1.0 / D)
                v = sq - mean * mean + EPS
                yi = jnp.full((16,), 0x5F3759DF, jnp.int32) - \
                    lax.shift_right_logical(plsc.bitcast(v, jnp.int32), 1)
                y = plsc.bitcast(yi, jnp.float32)
                vh = v * 0.5
                for _ in range(3):
                    y = y * (1.5 - vh * y * y)
                shift = y * (-mean)
                for k in range(NK):
                    t = xs[k] * y + shift
                    obuf[r, pl.ds(16 * k, 16)] = (
                        t * g_v[pl.ds(16 * k, 16)] + b_v[pl.ds(16 * k, 16)])
                return 0

            lax.fori_loop(0, CH, row_body, 0)

        gather(0, bufA, gsemA)
        gather(1, bufB, gsemB)

        def pair_body(g, _):
            c0 = 2 * g
            c1 = c0 + 1
            gather_wait(c0, bufA, gsemA)

            @pl.when(g != 0)
            def _():
                store_wait(c0 - 2, obufA, ssemA)

            compute(c0, bufA, obufA)

            @pl.when(g != n_pairs - 1)
            def _():
                gather(c0 + 2, bufA, gsemA)

            store(c0, obufA, ssemA)
            gather_wait(c1, bufB, gsemB)

            @pl.when(g != 0)
            def _():
                store_wait(c1 - 2, obufB, ssemB)

            compute(c1, bufB, obufB)

            @pl.when(g != n_pairs - 1)
            def _():
                gather(c1 + 2, bufB, gsemB)

            store(c1, obufB, ssemB)
            return 0

        lax.fori_loop(0, n_pairs, pair_body, 0)
        store_wait(n_chunks - 2, obufA, ssemA)
        store_wait(n_chunks - 1, obufB, ssemB)

    return fused


def kernel(input_ids, W, pos_table, gamma, beta):
    B, L = input_ids.shape
    V, D = W.shape
    N = B * L

    ids3d = input_ids.reshape(32, N // (32 * 128), 128).astype(jnp.int32)
    out = _make_fused(V, D, N, L)(W, ids3d, pos_table[:L], gamma, beta)
    return out.reshape(B, L, D)


# speedup vs baseline: 3.2234x; 3.2234x over previous
import functools

import jax
import jax.numpy as jnp
from jax import lax
from jax.experimental import pallas as pl
from jax.experimental.pallas import tpu as pltpu
from jax.experimental.pallas import tpu_sc as plsc

EPS = 1e-12


def _make_sc_gather(V, D, N):
    info = plsc.get_sparse_core_info()
    NW = info.num_cores * info.num_subcores
    CH = 128
    NB = 5
    assert N % (NW * CH) == 0
    n_chunks = N // (NW * CH)
    assert n_chunks % NB == 0

    mesh = plsc.VectorSubcoreMesh(core_axis_name="c", subcore_axis_name="s")

    @functools.partial(
        pl.kernel,
        mesh=mesh,
        out_type=jax.ShapeDtypeStruct((N, D), jnp.float32),
        scratch_types=[
            pltpu.VMEM((n_chunks, CH), jnp.int32),
            [pltpu.VMEM((CH, D), jnp.float32) for _ in range(NB)],
            [pltpu.SemaphoreType.DMA for _ in range(NB)],
            [pltpu.SemaphoreType.DMA for _ in range(NB)],
        ],
    )
    def gather_kernel(table_hbm, idx_hbm, out_hbm, idx_v, bufs, gsems, ssems):
        wid = lax.axis_index("s") * info.num_cores + lax.axis_index("c")
        pltpu.sync_copy(idx_hbm.at[wid], idx_v)

        def gather(c, b):
            pltpu.async_copy(table_hbm.at[idx_v.at[c]], bufs[b], gsems[b])

        def gather_wait(c, b):
            pltpu.make_async_copy(
                table_hbm.at[idx_v.at[c]], bufs[b], gsems[b]).wait()

        def store(c, b):
            pltpu.async_copy(
                bufs[b],
                out_hbm.at[pl.ds((wid * n_chunks + c) * CH, CH)], ssems[b])

        def store_wait(c, b):
            pltpu.make_async_copy(
                bufs[b],
                out_hbm.at[pl.ds((wid * n_chunks + c) * CH, CH)],
                ssems[b]).wait()

        for b in range(NB - 1):
            gather(b, b)
        for c in range(n_chunks):
            b = c % NB
            gather_wait(c, b)
            store(c, b)
            if c + NB - 1 < n_chunks:
                bn = (c + NB - 1) % NB
                if c - 1 >= 0:
                    store_wait(c - 1, (c - 1) % NB)
                gather(c + NB - 1, bn)
        for c in range(n_chunks - NB, n_chunks):
            store_wait(c, c % NB)

    return gather_kernel


def _ln_body(x_ref, pos_ref, g_ref, b_ref, o_ref):
    x = x_ref[...] + pos_ref[...][None, :, :]
    mean = jnp.mean(x, axis=-1, keepdims=True)
    xc = x - mean
    var = jnp.mean(xc * xc, axis=-1, keepdims=True)
    inv = lax.rsqrt(var + EPS)
    o_ref[...] = xc * inv * g_ref[0][None, None, :] + b_ref[0][None, None, :]


def _make_tc_ln(B, L, D, BB):
    return pl.pallas_call(
        _ln_body,
        out_shape=jax.ShapeDtypeStruct((B, L, D), jnp.float32),
        grid=(B // BB,),
        in_specs=[
            pl.BlockSpec((BB, L, D), lambda i: (i, 0, 0)),
            pl.BlockSpec((L, D), lambda i: (0, 0)),
            pl.BlockSpec((1, D), lambda i: (0, 0)),
            pl.BlockSpec((1, D), lambda i: (0, 0)),
        ],
        out_specs=pl.BlockSpec((BB, L, D), lambda i: (i, 0, 0)),
    )


def kernel(input_ids, W, pos_table, gamma, beta):
    B, L = input_ids.shape
    V, D = W.shape
    N = B * L

    ids3d = input_ids.reshape(32, N // (32 * 128), 128).astype(jnp.int32)
    gathered = _make_sc_gather(V, D, N)(W, ids3d)

    out = _make_tc_ln(B, L, D, BB=32)(
        gathered.reshape(B, L, D), pos_table[:L], gamma.reshape(1, D),
        beta.reshape(1, D))
    return out
